# baseline (speedup 1.0000x reference)
import jax
import jax.numpy as jnp
from jax import lax
from jax.experimental import pallas as pl
from jax.experimental.pallas import tpu as pltpu
from jax.experimental.pallas import tpu_sc as plsc

B = 16
N = 64
D = 768
H = 16
L = 3
NH = H * (L + 1)
NUM_ATOMS = 4608
NUM_IN = 512
NUM_OUT = 512
NUM_EDGES = 1536
NUM_SPATIAL = 512
MAX_DIST = 5
EF = 3
NS15 = MAX_DIST * EF
ER = NUM_EDGES + 1
RT = 1544
TR = MAX_DIST * RT
NC, NSUB, LANES = 2, 16, 16
NW = NC * NSUB
HC = NH // H
WPC = H // 2
NODE_K = 16
NPC = 2
CAT_ROWS = (NUM_ATOMS + 1) + NUM_IN + NUM_OUT


def _tables_body(e_ref, w_ref, o_ref):
    scale = jnp.float32(1.0 / EF)
    for d in range(MAX_DIST):
        o_ref[d] = jnp.dot(e_ref[...], w_ref[d],
                           preferred_element_type=jnp.float32) * scale


def _ln_body(rows_ref, gt_ref, g_ref, b_ref, o_ref):
    i = pl.program_id(0)

    def norm(x):
        mean = x.mean(axis=-1, keepdims=True)
        var = ((x - mean) ** 2).mean(axis=-1, keepdims=True)
        return (x - mean) * lax.rsqrt(var + 1e-5) * g_ref[...] + b_ref[...]

    @pl.when(i == 0)
    def _():
        o_ref[0] = jnp.broadcast_to(norm(gt_ref[...]), (B, D))

    @pl.when(i > 0)
    def _():
        rows = rows_ref[0].reshape(B, NODE_K, D).astype(jnp.float32)
        o_ref[0] = norm(rows.sum(axis=1))


def _sc_body(tbl_hbm, spat_hbm, eidx_hbm, spos_hbm, ab_hbm,
             nidx_hbm, cat_hbm,
             rows_out_hbm, gb_hbm,
             tbl_v, spat_v, nidx_v, rows_v,
             eidx_v, sp_v, ab_v, out_v,
             sg0, sg1, so0, so1, si0, si1, sb0, sb1):
    sem_g, sem_o = (sg0, sg1), (so0, so1)
    sem_in, sem_out = (si0, si1), (sb0, sb1)
    wid = lax.axis_index("s") * NC + lax.axis_index("c")
    hc = wid % HC
    g = wid // HC

    npt = B * N // NW
    node0 = wid * npt
    pltpu.sync_copy(nidx_hbm.at[pl.ds(node0 * NODE_K, npt * NODE_K)], nidx_v)
    gh = [None, None]
    oh = [None, None]

    def n_gather(ch):
        return pltpu.async_copy(
            cat_hbm.at[nidx_v.at[pl.ds(ch * NODE_K, NODE_K)]],
            rows_v.at[ch % 2], sem_g[ch % 2])

    gh[0] = n_gather(0)
    for ch in range(npt):
        p = ch % 2
        if ch + 1 < npt:
            if oh[(ch + 1) % 2] is not None:
                oh[(ch + 1) % 2].wait()
            gh[(ch + 1) % 2] = n_gather(ch + 1)
        gh[p].wait()
        oh[p] = pltpu.async_copy(
            rows_v.at[p],
            rows_out_hbm.at[pl.ds((node0 + ch) * NODE_K, NODE_K)], sem_o[p])
    oh[0].wait()
    oh[1].wait()

    pltpu.sync_copy(tbl_hbm.at[hc], tbl_v)
    pltpu.sync_copy(spat_hbm.at[hc], spat_v)

    ew = NS15 * 72
    h0 = hc * H

    def do_row(ri, eidx_b, sp_b, ab_b, out_b):
        for joff in (0, 16, 32, 48, 49):
            spv = sp_b[pl.ds(ri * 72 + joff, LANES)]
            s1 = jnp.maximum(spv, 1)
            s1 = s1 - jnp.where(s1 > 1, 1, 0)
            s1 = jnp.minimum(s1, MAX_DIST)
            recip = 1.0 / s1.astype(jnp.float32)
            abv = ab_b[pl.ds(ri * 72 + joff, LANES)]
            ab2 = abv + abv
            idxs = [eidx_b[pl.ds(ri * ew + s * 72 + joff, LANES)] * WPC
                    + ((s // EF) * RT * WPC) for s in range(NS15)]
            spv8 = spv * WPC
            for w in range(WPC):
                acc = jnp.zeros((2 * LANES,), jnp.bfloat16)
                for s in range(NS15):
                    gat = plsc.load_gather(tbl_v, [idxs[s] + w])
                    acc = acc + plsc.bitcast(gat, jnp.bfloat16)
                sg = plsc.load_gather(spat_v, [spv8 + w])
                ee_e, ee_o = plsc.unpack(
                    acc, format=plsc.PackFormat.INTERLEAVED)
                sp_e, sp_o = plsc.unpack(
                    plsc.bitcast(sg, jnp.bfloat16),
                    format=plsc.PackFormat.INTERLEAVED)
                out_b[2 * w, ri, pl.ds(joff, LANES)] = \
                    ee_e * recip + sp_e + ab2
                out_b[2 * w + 1, ri, pl.ds(joff, LANES)] = \
                    ee_o * recip + sp_o + ab2

    def blk(t):
        is_ep = t >= 16
        b = g * 2 + jnp.where(is_ep, t - 16, t // 8)
        i0 = jnp.where(is_ep, N, 8 * (t % 8))
        return b, i0, is_ep

    def issue_loads(t, p):
        b, i0, _ = blk(t)
        r0 = (b * 65 + i0)
        pltpu.async_copy(eidx_hbm.at[pl.ds(r0 * ew, 8 * ew)],
                         eidx_v.at[p], sem_in[p])
        pltpu.async_copy(spos_hbm.at[pl.ds(r0 * 72, 8 * 72)],
                         sp_v.at[p], sem_in[p])
        pltpu.async_copy(ab_hbm.at[pl.ds(r0 * 72, 8 * 72)],
                         ab_v.at[p], sem_in[p])

    def wait_loads(p):
        pltpu.make_async_copy(eidx_hbm.at[pl.ds(0, 8 * ew)],
                              eidx_v.at[p], sem_in[p]).wait()
        pltpu.make_async_copy(spos_hbm.at[pl.ds(0, 8 * 72)],
                              sp_v.at[p], sem_in[p]).wait()
        pltpu.make_async_copy(ab_hbm.at[pl.ds(0, 8 * 72)],
                              ab_v.at[p], sem_in[p]).wait()

    def out_dst(t):
        b, i0, _ = blk(t)
        return gb_hbm.at[b, pl.ds(h0, H), pl.ds(i0, 8)]

    def half(m, t, p):
        @pl.when(t + 1 < 18)
        def _():
            issue_loads(t + 1, 1 - p)
        wait_loads(p)

        @pl.when(m >= 1)
        def _():
            pltpu.make_async_copy(out_v.at[p], out_dst(t - 2),
                                  sem_out[p]).wait()
        lax.fori_loop(
            0, 8, lambda ri, c: (do_row(ri, eidx_v.at[p], sp_v.at[p],
                                        ab_v.at[p], out_v.at[p]), c)[1], 0)

        @pl.when(t < 16)
        def _():
            pltpu.async_copy(out_v.at[p], out_dst(t), sem_out[p])

        @pl.when(t >= 16)
        def _():
            b, _, _ = blk(t)
            pltpu.async_copy(out_v.at[p, :, pl.ds(0, 1)],
                             gb_hbm.at[b, pl.ds(h0, H), pl.ds(N, 1)],
                             sem_out[p])

    issue_loads(jnp.int32(0), 0)

    def pair_body(m, carry):
        half(m, 2 * m, 0)
        half(m, 2 * m + 1, 1)
        return carry

    lax.fori_loop(0, 9, pair_body, 0)

    for p in range(2):
        b = g * 2 + p
        pltpu.make_async_copy(out_v.at[p, :, pl.ds(0, 1)],
                              gb_hbm.at[b, pl.ds(h0, H), pl.ds(N, 1)],
                              sem_out[p]).wait()


def _run_sc(tbl_cm, spat_cm, eidx_t, spatial_pos, ab_pad, node_idx,
            cat_tbl):
    mesh = plsc.VectorSubcoreMesh(core_axis_name="c", subcore_axis_name="s",
                                  num_cores=NC, num_subcores=NSUB)
    f = pl.kernel(
        _sc_body,
        out_type=(
            jax.ShapeDtypeStruct((B * N * NODE_K, D // 2), jnp.int32),
            jax.ShapeDtypeStruct((B, NH, N + 1, N + 1), jnp.float32),
        ),
        mesh=mesh,
        compiler_params=pltpu.CompilerParams(needs_layout_passes=False,
                                             use_tc_tiling_on_sc=False),
        scratch_types=[
            pltpu.VMEM((TR * WPC,), jnp.int32),
            pltpu.VMEM((520 * WPC,), jnp.int32),
            pltpu.VMEM((B * N * NODE_K // NW,), jnp.int32),
            pltpu.VMEM((2, NODE_K, D // 2), jnp.int32),
            pltpu.VMEM((2, 8 * NS15 * 72), jnp.int32),
            pltpu.VMEM((2, 8 * 72), jnp.int32),
            pltpu.VMEM((2, 8 * 72), jnp.float32),
            pltpu.VMEM((2, H, 8, N + 1), jnp.float32),
        ] + [pltpu.SemaphoreType.DMA] * 8,
    )
    return f(tbl_cm, spat_cm, eidx_t, spatial_pos, ab_pad, node_idx,
             cat_tbl)


def kernel(input_ids, llm_mask, dummy, x_0, in_degree, out_degree, attn_bias,
           spatial_pos, edge_input, num_atoms, pos, mask3d_filter,
           node_type_edge, atom_emb, in_deg_emb, out_deg_emb, graph_token,
           spatial_emb, edge_emb, edge_dis_emb, graph_token_vd, ln_gamma,
           ln_beta):
    w5 = edge_dis_emb.reshape(-1, NH, NH)[:MAX_DIST]
    t_f32 = pl.pallas_call(
        _tables_body,
        out_shape=jax.ShapeDtypeStruct((MAX_DIST, ER, NH), jnp.float32),
    )(edge_emb, w5)
    t_pad = jnp.pad(t_f32, ((0, 0), (0, RT - ER), (0, 0)))
    t_u32 = lax.bitcast_convert_type(
        t_pad.astype(jnp.bfloat16).reshape(TR, WPC * HC, 2), jnp.int32)
    tbl_cm = (t_u32.reshape(TR, HC, WPC).transpose(1, 0, 2)
              .reshape(HC, TR * WPC))
    spat2 = jnp.concatenate(
        [spatial_emb, graph_token_vd.reshape(1, NH)], axis=0)
    s_u32 = lax.bitcast_convert_type(
        spat2.astype(jnp.bfloat16).reshape(NUM_SPATIAL + 1, WPC * HC, 2),
        jnp.int32)
    s_u32 = jnp.pad(s_u32, ((0, 520 - (NUM_SPATIAL + 1)), (0, 0)))
    spat_cm = (s_u32.reshape(520, HC, WPC).transpose(1, 0, 2)
               .reshape(HC, 520 * WPC))

    eidx_t = edge_input.reshape(B, N, N, NS15).transpose(0, 1, 3, 2)
    eidx_p = jnp.full((B, 65, NS15, 72), ER, jnp.int32)
    eidx_p = eidx_p.at[:, 1:, :, 1:65].set(eidx_t)
    eidx_t = jnp.pad(eidx_p.reshape(-1), (0, 7 * NS15 * 72))
    sp_pad = jnp.full((B, 65, 72), NUM_SPATIAL, jnp.int32)
    sp_pad = jnp.pad(sp_pad.at[:, 1:, 1:65].set(spatial_pos).reshape(-1),
                     (0, 7 * 72))
    ab_pad = jnp.pad(attn_bias,
                     ((0, 0), (0, 0), (0, 72 - (N + 1)))).reshape(-1)
    ab_pad = jnp.pad(ab_pad, (0, 7 * 72))
    nidx = jnp.concatenate(
        [x_0, in_degree[..., None] + (NUM_ATOMS + 1),
         out_degree[..., None] + (NUM_ATOMS + 1 + NUM_IN)], axis=-1)
    nidx = jnp.pad(nidx, ((0, 0), (0, 0), (0, NODE_K - 11)))
    nidx = nidx.transpose(1, 0, 2).reshape(-1)
    cat_tbl = jnp.concatenate([atom_emb, in_deg_emb, out_deg_emb],
                              axis=0).astype(jnp.bfloat16)
    cat_tbl = lax.bitcast_convert_type(
        cat_tbl.reshape(CAT_ROWS, D // 2, 2), jnp.int32)

    node_rows, gb = _run_sc(tbl_cm, spat_cm, eidx_t, sp_pad, ab_pad,
                            nidx, cat_tbl)

    x = pl.pallas_call(
        _ln_body,
        grid=(N + 1,),
        in_specs=[
            pl.BlockSpec((1, B * NODE_K, D),
                         lambda i: (jnp.maximum(i - 1, 0), 0, 0)),
            pl.BlockSpec((1, D), lambda i: (0, 0)),
            pl.BlockSpec((1, D), lambda i: (0, 0)),
            pl.BlockSpec((1, D), lambda i: (0, 0)),
        ],
        out_specs=pl.BlockSpec((1, B, D), lambda i: (i, 0, 0)),
        out_shape=jax.ShapeDtypeStruct((N + 1, B, D), jnp.float32),
    )(lax.bitcast_convert_type(node_rows, jnp.bfloat16)
      .reshape(N, B * NODE_K, D), graph_token,
      ln_gamma.reshape(1, D), ln_beta.reshape(1, D))

    padding_mask = jnp.concatenate(
        [jnp.zeros((B, 1), dtype=bool), x_0[:, :, 0] == 0], axis=1)
    attn_bias_out = gb.reshape(B, L + 1, H, N + 1, N + 1)
    return (x, padding_mask, attn_bias_out, input_ids, llm_mask)


# speedup vs baseline: 1.1833x; 1.1833x over previous
import jax
import jax.numpy as jnp
from jax import lax
from jax.experimental import pallas as pl
from jax.experimental.pallas import tpu as pltpu
from jax.experimental.pallas import tpu_sc as plsc

B = 16
N = 64
D = 768
H = 16
L = 3
NH = H * (L + 1)
NUM_ATOMS = 4608
NUM_IN = 512
NUM_OUT = 512
NUM_EDGES = 1536
NUM_SPATIAL = 512
MAX_DIST = 5
EF = 3
NS15 = MAX_DIST * EF
ER = NUM_EDGES + 1
RT = 1544
TR = MAX_DIST * RT
NC, NSUB, LANES = 2, 16, 16
NW = NC * NSUB
HC = NH // H
WPC = H // 2
NODE_K = 16
NPC = 2
CAT_ROWS = (NUM_ATOMS + 1) + NUM_IN + NUM_OUT


def _tables_body(e_ref, w_ref, o_ref):
    scale = jnp.float32(1.0 / EF)
    for d in range(MAX_DIST):
        o_ref[d] = jnp.dot(e_ref[...], w_ref[d],
                           preferred_element_type=jnp.float32) * scale


def _ln_body(rows_ref, gt_ref, g_ref, b_ref, o_ref):
    i = pl.program_id(0)

    def norm(x):
        mean = x.mean(axis=-1, keepdims=True)
        var = ((x - mean) ** 2).mean(axis=-1, keepdims=True)
        return (x - mean) * lax.rsqrt(var + 1e-5) * g_ref[...] + b_ref[...]

    @pl.when(i == 0)
    def _():
        o_ref[0] = jnp.broadcast_to(norm(gt_ref[...]), (B, D))

    @pl.when(i > 0)
    def _():
        rows = rows_ref[0].reshape(B, NODE_K, D).astype(jnp.float32)
        o_ref[0] = norm(rows.sum(axis=1))


def _sc_body(tbl_hbm, spat_hbm, eidx_hbm, spos_hbm, ab_hbm,
             nidx_hbm, cat_hbm,
             rows_out_hbm, gb_hbm,
             tbl_v, spat_v, nidx_v, rows_v,
             eidx_v, sp_v, ab_v, out_v,
             sg0, sg1, so0, so1, si0, si1, sb0, sb1):
    sem_g, sem_o = (sg0, sg1), (so0, so1)
    sem_in, sem_out = (si0, si1), (sb0, sb1)
    wid = lax.axis_index("s") * NC + lax.axis_index("c")
    hc = wid % HC
    g = wid // HC

    npt = B * N // NW
    node0 = wid * npt
    pltpu.sync_copy(nidx_hbm.at[pl.ds(node0 * NODE_K, npt * NODE_K)], nidx_v)
    gh = [None, None]
    oh = [None, None]

    def n_gather(ch):
        return pltpu.async_copy(
            cat_hbm.at[nidx_v.at[pl.ds(ch * NODE_K, NODE_K)]],
            rows_v.at[ch % 2], sem_g[ch % 2])

    gh[0] = n_gather(0)
    for ch in range(npt):
        p = ch % 2
        if ch + 1 < npt:
            if oh[(ch + 1) % 2] is not None:
                oh[(ch + 1) % 2].wait()
            gh[(ch + 1) % 2] = n_gather(ch + 1)
        gh[p].wait()
        oh[p] = pltpu.async_copy(
            rows_v.at[p],
            rows_out_hbm.at[pl.ds((node0 + ch) * NODE_K, NODE_K)], sem_o[p])
    oh[0].wait()
    oh[1].wait()

    pltpu.sync_copy(tbl_hbm.at[hc], tbl_v)
    pltpu.sync_copy(spat_hbm.at[hc], spat_v)

    ew = NS15 * 72
    h0 = hc * H

    def do_row(ri, eidx_b, sp_b, ab_b, out_b):
        for joff in (0, 16, 32, 48, 49):
            spv = sp_b[pl.ds(ri * 72 + joff, LANES)]
            s1 = jnp.maximum(spv, 1)
            s1 = s1 - jnp.where(s1 > 1, 1, 0)
            s1 = jnp.minimum(s1, MAX_DIST)
            recip = 1.0 / s1.astype(jnp.float32)
            abv = ab_b[pl.ds(ri * 72 + joff, LANES)]
            ab2 = abv + abv
            idxs = [eidx_b[pl.ds(ri * ew + s * 72 + joff, LANES)] * WPC
                    + ((s // EF) * RT * WPC) for s in range(NS15)]
            spv8 = spv * WPC
            for w in range(WPC):
                gs = [plsc.bitcast(plsc.load_gather(tbl_v, [idxs[s] + w]),
                                   jnp.bfloat16) for s in range(NS15)]
                while len(gs) > 1:
                    gs = [a + b for a, b in zip(gs[::2], gs[1::2])] \
                        + ([gs[-1]] if len(gs) % 2 else [])
                acc = gs[0]
                sg = plsc.load_gather(spat_v, [spv8 + w])
                ee_e, ee_o = plsc.unpack(
                    acc, format=plsc.PackFormat.INTERLEAVED)
                sp_e, sp_o = plsc.unpack(
                    plsc.bitcast(sg, jnp.bfloat16),
                    format=plsc.PackFormat.INTERLEAVED)
                out_b[2 * w, ri, pl.ds(joff, LANES)] = \
                    ee_e * recip + sp_e + ab2
                out_b[2 * w + 1, ri, pl.ds(joff, LANES)] = \
                    ee_o * recip + sp_o + ab2

    def blk(t):
        is_ep = t >= 16
        b = g * 2 + jnp.where(is_ep, t - 16, t // 8)
        i0 = jnp.where(is_ep, N, 8 * (t % 8))
        return b, i0, is_ep

    def issue_loads(t, p):
        b, i0, _ = blk(t)
        r0 = (b * 65 + i0)
        pltpu.async_copy(eidx_hbm.at[pl.ds(r0 * ew, 8 * ew)],
                         eidx_v.at[p], sem_in[p])
        pltpu.async_copy(spos_hbm.at[pl.ds(r0 * 72, 8 * 72)],
                         sp_v.at[p], sem_in[p])
        pltpu.async_copy(ab_hbm.at[pl.ds(r0 * 72, 8 * 72)],
                         ab_v.at[p], sem_in[p])

    def wait_loads(p):
        pltpu.make_async_copy(eidx_hbm.at[pl.ds(0, 8 * ew)],
                              eidx_v.at[p], sem_in[p]).wait()
        pltpu.make_async_copy(spos_hbm.at[pl.ds(0, 8 * 72)],
                              sp_v.at[p], sem_in[p]).wait()
        pltpu.make_async_copy(ab_hbm.at[pl.ds(0, 8 * 72)],
                              ab_v.at[p], sem_in[p]).wait()

    def out_dst(t):
        b, i0, _ = blk(t)
        return gb_hbm.at[b, pl.ds(h0, H), pl.ds(i0, 8)]

    def half(m, t, p):
        @pl.when(t + 1 < 18)
        def _():
            issue_loads(t + 1, 1 - p)
        wait_loads(p)

        @pl.when(m >= 1)
        def _():
            pltpu.make_async_copy(out_v.at[p], out_dst(t - 2),
                                  sem_out[p]).wait()
        lax.fori_loop(
            0, 8, lambda ri, c: (do_row(ri, eidx_v.at[p], sp_v.at[p],
                                        ab_v.at[p], out_v.at[p]), c)[1], 0)

        @pl.when(t < 16)
        def _():
            pltpu.async_copy(out_v.at[p], out_dst(t), sem_out[p])

        @pl.when(t >= 16)
        def _():
            b, _, _ = blk(t)
            pltpu.async_copy(out_v.at[p, :, pl.ds(0, 1)],
                             gb_hbm.at[b, pl.ds(h0, H), pl.ds(N, 1)],
                             sem_out[p])

    issue_loads(jnp.int32(0), 0)

    def pair_body(m, carry):
        half(m, 2 * m, 0)
        half(m, 2 * m + 1, 1)
        return carry

    lax.fori_loop(0, 9, pair_body, 0)

    for p in range(2):
        b = g * 2 + p
        pltpu.make_async_copy(out_v.at[p, :, pl.ds(0, 1)],
                              gb_hbm.at[b, pl.ds(h0, H), pl.ds(N, 1)],
                              sem_out[p]).wait()


def _run_sc(tbl_cm, spat_cm, eidx_t, spatial_pos, ab_pad, node_idx,
            cat_tbl):
    mesh = plsc.VectorSubcoreMesh(core_axis_name="c", subcore_axis_name="s",
                                  num_cores=NC, num_subcores=NSUB)
    f = pl.kernel(
        _sc_body,
        out_type=(
            jax.ShapeDtypeStruct((B * N * NODE_K, D // 2), jnp.int32),
            jax.ShapeDtypeStruct((B, NH, N + 1, N + 1), jnp.float32),
        ),
        mesh=mesh,
        compiler_params=pltpu.CompilerParams(needs_layout_passes=False,
                                             use_tc_tiling_on_sc=False),
        scratch_types=[
            pltpu.VMEM((TR * WPC,), jnp.int32),
            pltpu.VMEM((520 * WPC,), jnp.int32),
            pltpu.VMEM((B * N * NODE_K // NW,), jnp.int32),
            pltpu.VMEM((2, NODE_K, D // 2), jnp.int32),
            pltpu.VMEM((2, 8 * NS15 * 72), jnp.int32),
            pltpu.VMEM((2, 8 * 72), jnp.int32),
            pltpu.VMEM((2, 8 * 72), jnp.float32),
            pltpu.VMEM((2, H, 8, N + 1), jnp.float32),
        ] + [pltpu.SemaphoreType.DMA] * 8,
    )
    return f(tbl_cm, spat_cm, eidx_t, spatial_pos, ab_pad, node_idx,
             cat_tbl)


def kernel(input_ids, llm_mask, dummy, x_0, in_degree, out_degree, attn_bias,
           spatial_pos, edge_input, num_atoms, pos, mask3d_filter,
           node_type_edge, atom_emb, in_deg_emb, out_deg_emb, graph_token,
           spatial_emb, edge_emb, edge_dis_emb, graph_token_vd, ln_gamma,
           ln_beta):
    w5 = edge_dis_emb.reshape(-1, NH, NH)[:MAX_DIST]
    t_f32 = pl.pallas_call(
        _tables_body,
        out_shape=jax.ShapeDtypeStruct((MAX_DIST, ER, NH), jnp.float32),
    )(edge_emb, w5)
    t_pad = jnp.pad(t_f32, ((0, 0), (0, RT - ER), (0, 0)))
    t_u32 = lax.bitcast_convert_type(
        t_pad.astype(jnp.bfloat16).reshape(TR, WPC * HC, 2), jnp.int32)
    tbl_cm = (t_u32.reshape(TR, HC, WPC).transpose(1, 0, 2)
              .reshape(HC, TR * WPC))
    spat2 = jnp.concatenate(
        [spatial_emb, graph_token_vd.reshape(1, NH)], axis=0)
    s_u32 = lax.bitcast_convert_type(
        spat2.astype(jnp.bfloat16).reshape(NUM_SPATIAL + 1, WPC * HC, 2),
        jnp.int32)
    s_u32 = jnp.pad(s_u32, ((0, 520 - (NUM_SPATIAL + 1)), (0, 0)))
    spat_cm = (s_u32.reshape(520, HC, WPC).transpose(1, 0, 2)
               .reshape(HC, 520 * WPC))

    eidx_t = edge_input.reshape(B, N, N, NS15).transpose(0, 1, 3, 2)
    eidx_p = jnp.full((B, 65, NS15, 72), ER, jnp.int32)
    eidx_p = eidx_p.at[:, 1:, :, 1:65].set(eidx_t)
    eidx_t = jnp.pad(eidx_p.reshape(-1), (0, 7 * NS15 * 72))
    sp_pad = jnp.full((B, 65, 72), NUM_SPATIAL, jnp.int32)
    sp_pad = jnp.pad(sp_pad.at[:, 1:, 1:65].set(spatial_pos).reshape(-1),
                     (0, 7 * 72))
    ab_pad = jnp.pad(attn_bias,
                     ((0, 0), (0, 0), (0, 72 - (N + 1)))).reshape(-1)
    ab_pad = jnp.pad(ab_pad, (0, 7 * 72))
    nidx = jnp.concatenate(
        [x_0, in_degree[..., None] + (NUM_ATOMS + 1),
         out_degree[..., None] + (NUM_ATOMS + 1 + NUM_IN)], axis=-1)
    nidx = jnp.pad(nidx, ((0, 0), (0, 0), (0, NODE_K - 11)))
    nidx = nidx.transpose(1, 0, 2).reshape(-1)
    cat_tbl = jnp.concatenate([atom_emb, in_deg_emb, out_deg_emb],
                              axis=0).astype(jnp.bfloat16)
    cat_tbl = lax.bitcast_convert_type(
        cat_tbl.reshape(CAT_ROWS, D // 2, 2), jnp.int32)

    node_rows, gb = _run_sc(tbl_cm, spat_cm, eidx_t, sp_pad, ab_pad,
                            nidx, cat_tbl)

    x = pl.pallas_call(
        _ln_body,
        grid=(N + 1,),
        in_specs=[
            pl.BlockSpec((1, B * NODE_K, D),
                         lambda i: (jnp.maximum(i - 1, 0), 0, 0)),
            pl.BlockSpec((1, D), lambda i: (0, 0)),
            pl.BlockSpec((1, D), lambda i: (0, 0)),
            pl.BlockSpec((1, D), lambda i: (0, 0)),
        ],
        out_specs=pl.BlockSpec((1, B, D), lambda i: (i, 0, 0)),
        out_shape=jax.ShapeDtypeStruct((N + 1, B, D), jnp.float32),
    )(lax.bitcast_convert_type(node_rows, jnp.bfloat16)
      .reshape(N, B * NODE_K, D), graph_token,
      ln_gamma.reshape(1, D), ln_beta.reshape(1, D))

    padding_mask = jnp.concatenate(
        [jnp.zeros((B, 1), dtype=bool), x_0[:, :, 0] == 0], axis=1)
    attn_bias_out = gb.reshape(B, L + 1, H, N + 1, N + 1)
    return (x, padding_mask, attn_bias_out, input_ids, llm_mask)
